# Initial kernel scaffold; baseline (speedup 1.0000x reference)
import functools

import jax
import jax.numpy as jnp
from jax import lax
from jax.experimental import pallas as pl
from jax.experimental.pallas import tpu as pltpu
from jax.experimental.pallas import tpu_sc as plsc

NN = 50000
EE = 800000
HH = 48
NR = 4
NH = 4
DH = HH // NH
FH = HH // 2
CH = 128
NCHUNK = EE // CH
BS = 1000
NB = NN // BS
CNT = NR * NN

_prec = jax.lax.Precision.HIGHEST


def _dot(a, b):
    return jax.lax.dot_general(a, b, (((a.ndim - 1,), (0,)), ((), ())),
                               precision=_prec, preferred_element_type=jnp.float32)


_sc_mesh = plsc.VectorSubcoreMesh(core_axis_name="c", subcore_axis_name="s")


@functools.partial(
    pl.kernel,
    out_type=jax.ShapeDtypeStruct((2, CNT), jnp.float32),
    mesh=_sc_mesh,
    scratch_types=[
        pltpu.VMEM_SHARED((CNT,), jnp.float32),
        pltpu.VMEM((CH,), jnp.int32),
        pltpu.VMEM((CH,), jnp.int32),
        pltpu.VMEM((CH,), jnp.int32),
        pltpu.VMEM((CH,), jnp.float32),
    ],
)
def _sc_counts(dst_hbm, typ_hbm, zeros_hbm, cnt_hbm, cnt_sh, dv, tv, cev, ones_v):
    c = lax.axis_index("c")
    s = lax.axis_index("s")
    w = c * 16 + s

    def zinit(k, carry):
        bid = s + k * 16
        @pl.when(bid < CNT // 1000)
        def _():
            pltpu.sync_copy(zeros_hbm, cnt_sh.at[pl.ds(bid * 1000, 1000)])
        return carry
    lax.fori_loop(0, (CNT // 1000 + 15) // 16, zinit, 0)

    for g in range(CH // 16):
        ones_v[pl.ds(g * 16, 16)] = jnp.full((16,), 1.0, jnp.float32)
    plsc.subcore_barrier()

    def chunk(k, carry):
        cid = w + k * 32
        @pl.when(cid < NCHUNK)
        def _():
            base = cid * CH
            pltpu.sync_copy(dst_hbm.at[pl.ds(base, CH)], dv)
            pltpu.sync_copy(typ_hbm.at[pl.ds(base, CH)], tv)
            for g in range(CH // 16):
                sl = pl.ds(g * 16, 16)
                cev[sl] = tv[sl] * NN + dv[sl]
            pltpu.sync_copy(ones_v, cnt_sh.at[cev], add=True)
        return carry
    lax.fori_loop(0, (NCHUNK + 31) // 32, chunk, 0)
    plsc.subcore_barrier()

    def outb(k, carry):
        bid = s + k * 16
        @pl.when(bid < CNT // 1000)
        def _():
            pltpu.sync_copy(cnt_sh.at[pl.ds(bid * 1000, 1000)],
                            cnt_hbm.at[c].at[pl.ds(bid * 1000, 1000)])
        return carry
    lax.fori_loop(0, (CNT // 1000 + 15) // 16, outb, 0)


@functools.partial(
    pl.kernel,
    out_type=jax.ShapeDtypeStruct((2 * NN, FH), jnp.float32),
    mesh=_sc_mesh,
    scratch_types=[
        pltpu.VMEM_SHARED((NN, FH), jnp.float32),
        pltpu.VMEM((CH,), jnp.int32),
        pltpu.VMEM((CH,), jnp.int32),
        pltpu.VMEM((CH,), jnp.int32),
        pltpu.VMEM((CH,), jnp.int32),
        pltpu.VMEM((CH,), jnp.int32),
        pltpu.VMEM((CH,), jnp.float32),
        pltpu.VMEM((CH, FH), jnp.float32),
        pltpu.SemaphoreType.DMA,
    ],
)
def _sc_edges(src_hbm, dst_hbm, typ_hbm, y_hbm, inv_hbm, dinit_hbm, o_hbm,
              acc, sv, dv, tv, ridx, cev, w_v, rows, sem):
    c = lax.axis_index("c")
    s = lax.axis_index("s")

    def initb(k, carry):
        bid = s + k * 16
        @pl.when(bid < NB)
        def _():
            pltpu.sync_copy(dinit_hbm.at[pl.ds(c * NN + bid * BS, BS)],
                            acc.at[pl.ds(bid * BS, BS)])
        return carry
    lax.fori_loop(0, (NB + 15) // 16, initb, 0)
    plsc.subcore_barrier()

    iota = lax.iota(jnp.int32, 16)

    def chunk(k, carry):
        cid = s + k * 16
        @pl.when(cid < NCHUNK)
        def _():
            base = cid * CH
            pltpu.sync_copy(src_hbm.at[pl.ds(base, CH)], sv)
            pltpu.sync_copy(dst_hbm.at[pl.ds(base, CH)], dv)
            pltpu.sync_copy(typ_hbm.at[pl.ds(base, CH)], tv)
            off = c * (NR * NN)
            for g in range(CH // 16):
                sl = pl.ds(g * 16, 16)
                t16 = tv[sl]
                ridx[sl] = off + t16 * NN + sv[sl]
                cev[sl] = t16 * NN + dv[sl]
            pltpu.async_copy(inv_hbm.at[cev], w_v, sem).wait()
            pltpu.async_copy(y_hbm.at[ridx], rows, sem).wait()
            for g in range(CH // 16):
                w16 = w_v[pl.ds(g * 16, 16)]
                rid = g * 16 + iota
                for j in range(FH):
                    cj = jnp.full((16,), j, jnp.int32)
                    v = plsc.load_gather(rows, [rid, cj])
                    plsc.store_scatter(rows, [rid, cj], v * w16)
            pltpu.sync_copy(rows, acc.at[dv], add=True)
        return carry
    lax.fori_loop(0, (NCHUNK + 15) // 16, chunk, 0)
    plsc.subcore_barrier()

    def outb(k, carry):
        bid = s + k * 16
        @pl.when(bid < NB)
        def _():
            pltpu.sync_copy(acc.at[pl.ds(bid * BS, BS)],
                            o_hbm.at[pl.ds(c * NN + bid * BS, BS)])
        return carry
    lax.fori_loop(0, (NB + 15) // 16, outb, 0)


def _tc1_body(meta_ref, text_ref, mW1, mb1, mW2, mb2, tW1, tb1, tW2, tb2,
              g1W, g1root, g1b, cnt_ref,
              metaf_ref, textf_ref, y_ref, d_ref, inv_ref):
    mf = jnp.maximum(_dot(meta_ref[...], mW1[...].T) + mb1[...], 0.0)
    mf = jnp.maximum(_dot(mf, mW2[...].T) + mb2[...], 0.0)
    tf = jnp.maximum(_dot(text_ref[...], tW1[...].T) + tb1[...], 0.0)
    tf = jnp.maximum(_dot(tf, tW2[...].T) + tb2[...], 0.0)
    metaf_ref[...] = mf
    textf_ref[...] = tf
    for r in range(NR):
        yr = _dot(mf, g1W[r, :HH, :]) + _dot(tf, g1W[r, HH:, :])
        y_ref[0, r] = yr[:, :FH]
        y_ref[1, r] = yr[:, FH:]
    d = _dot(mf, g1root[:HH, :]) + _dot(tf, g1root[HH:, :]) + g1b[...]
    d_ref[0] = d[:, :FH]
    d_ref[1] = d[:, FH:]
    csum = cnt_ref[0, 0] + cnt_ref[1, 0]
    inv_ref[0] = 1.0 / jnp.maximum(csum, 1.0)


def _tc2_body(o1_ref, g2W, g2root, g2b, y_ref, d_ref):
    ra = jnp.maximum(o1_ref[0], 0.0)
    rb = jnp.maximum(o1_ref[1], 0.0)
    for r in range(NR):
        yr = _dot(ra, g2W[r, :FH, :]) + _dot(rb, g2W[r, FH:, :])
        y_ref[0, r] = yr[:, :FH]
        y_ref[1, r] = yr[:, FH:]
    d = _dot(ra, g2root[:FH, :]) + _dot(rb, g2root[FH:, :]) + g2b[...]
    d_ref[0] = d[:, :FH]
    d_ref[1] = d[:, FH:]


def _tc3_body(o2_ref, textf_ref, metaf_ref, Wi, bi, Wo, bo, conWT, con_b,
              out_ref):
    ga = jnp.maximum(o2_ref[0], 0.0)
    gb = jnp.maximum(o2_ref[1], 0.0)
    tf = textf_ref[...]
    mf = metaf_ref[...]

    def proj(lo):
        M = Wi[...][lo:lo + HH, :].T
        b = bi[...][:, lo:lo + HH]
        qg = _dot(ga, M[:FH, :]) + _dot(gb, M[FH:, :]) + b
        qt = _dot(tf, M) + b
        qm = _dot(mf, M) + b
        return (qg, qt, qm)

    q = proj(0)
    kk = proj(HH)
    v = proj(2 * HH)

    hsel = (jax.lax.broadcasted_iota(jnp.int32, (HH, NH), 0) // DH ==
            jax.lax.broadcasted_iota(jnp.int32, (HH, NH), 1)).astype(jnp.float32)
    scale = 1.0 / (DH ** 0.5)

    s = [[_dot(q[i] * kk[j], hsel) * scale for j in range(3)] for i in range(3)]
    a = []
    aw = []
    for i in range(3):
        m = jnp.maximum(jnp.maximum(s[i][0], s[i][1]), s[i][2])
        e = [jnp.exp(s[i][j] - m) for j in range(3)]
        z = e[0] + e[1] + e[2]
        ai = [e[j] / z for j in range(3)]
        a.append(ai)
        aw.append([jnp.sum(ai[j], axis=1, keepdims=True) * (1.0 / NH)
                   for j in range(3)])

    hselT = hsel.T
    WoT = Wo[...].T
    f_out = []
    for i in range(3):
        oi = (_dot(a[i][0], hselT) * v[0] +
              _dot(a[i][1], hselT) * v[1] +
              _dot(a[i][2], hselT) * v[2])
        f_out.append(_dot(oi, WoT) + bo[...])

    fc = con_b[...]
    for i in range(3):
        for j in range(3):
            fc = fc + aw[i][j] * conWT[3 * i + j:3 * i + j + 1, :]

    out_ref[:, 0:HH] = f_out[0]
    out_ref[:, HH:2 * HH] = f_out[1]
    out_ref[:, 2 * HH:3 * HH] = f_out[2]
    out_ref[:, 3 * HH:4 * HH] = fc


def _full_spec(shape):
    return pl.BlockSpec(shape, lambda *args: tuple(0 for _ in shape))


def kernel(meta, text, edge_index, edge_type, meta_W1, meta_b1, meta_W2, meta_b2,
           text_W1, text_b1, text_W2, text_b2, g1_W, g1_root, g1_b,
           g2_W, g2_root, g2_b, attn_Wi, attn_bi, attn_Wo, attn_bo, con_W, con_b):
    f32 = jnp.float32
    src = edge_index[0]
    dst = edge_index[1]
    typ = edge_type
    zeros1k = jnp.zeros((1000,), f32)

    cnt = _sc_counts(dst, typ, zeros1k)

    row2 = lambda x: x.reshape(1, -1)
    tc1 = pl.pallas_call(
        _tc1_body,
        grid=(NB,),
        in_specs=[
            pl.BlockSpec((BS, 16), lambda i: (i, 0)),
            pl.BlockSpec((BS, 768), lambda i: (i, 0)),
            _full_spec((HH, 16)), _full_spec((1, HH)),
            _full_spec((HH, HH)), _full_spec((1, HH)),
            _full_spec((HH, 768)), _full_spec((1, HH)),
            _full_spec((HH, HH)), _full_spec((1, HH)),
            _full_spec((NR, 2 * HH, HH)), _full_spec((2 * HH, HH)),
            _full_spec((1, HH)),
            pl.BlockSpec((2, 1, 1, CNT // NB), lambda i: (0, i, 0, 0)),
        ],
        out_specs=[
            pl.BlockSpec((BS, HH), lambda i: (i, 0)),
            pl.BlockSpec((BS, HH), lambda i: (i, 0)),
            pl.BlockSpec((2, NR, BS, FH), lambda i: (0, 0, i, 0)),
            pl.BlockSpec((2, BS, FH), lambda i: (0, i, 0)),
            pl.BlockSpec((1, 1, CNT // NB), lambda i: (i, 0, 0)),
        ],
        out_shape=[
            jax.ShapeDtypeStruct((NN, HH), f32),
            jax.ShapeDtypeStruct((NN, HH), f32),
            jax.ShapeDtypeStruct((2, NR, NN, FH), f32),
            jax.ShapeDtypeStruct((2, NN, FH), f32),
            jax.ShapeDtypeStruct((NB, 1, CNT // NB), f32),
        ],
    )
    meta_f, text_f, y1, d1, inv = tc1(
        meta, text, meta_W1, row2(meta_b1), meta_W2, row2(meta_b2),
        text_W1, row2(text_b1), text_W2, row2(text_b2),
        g1_W, g1_root, row2(g1_b), cnt.reshape(2, NB, 1, CNT // NB))

    inv_flat = inv.reshape(CNT)
    o1 = _sc_edges(src, dst, typ, y1.reshape(2 * NR * NN, FH), inv_flat,
                   d1.reshape(2 * NN, FH))

    tc2 = pl.pallas_call(
        _tc2_body,
        grid=(NB,),
        in_specs=[
            pl.BlockSpec((2, BS, FH), lambda i: (0, i, 0)),
            _full_spec((NR, HH, HH)), _full_spec((HH, HH)), _full_spec((1, HH)),
        ],
        out_specs=[
            pl.BlockSpec((2, NR, BS, FH), lambda i: (0, 0, i, 0)),
            pl.BlockSpec((2, BS, FH), lambda i: (0, i, 0)),
        ],
        out_shape=[
            jax.ShapeDtypeStruct((2, NR, NN, FH), f32),
            jax.ShapeDtypeStruct((2, NN, FH), f32),
        ],
    )
    y2, d2 = tc2(o1.reshape(2, NN, FH), g2_W, g2_root, row2(g2_b))

    o2 = _sc_edges(src, dst, typ, y2.reshape(2 * NR * NN, FH), inv_flat,
                   d2.reshape(2 * NN, FH))

    tc3 = pl.pallas_call(
        _tc3_body,
        grid=(NB,),
        in_specs=[
            pl.BlockSpec((2, BS, FH), lambda i: (0, i, 0)),
            pl.BlockSpec((BS, HH), lambda i: (i, 0)),
            pl.BlockSpec((BS, HH), lambda i: (i, 0)),
            _full_spec((3 * HH, HH)), _full_spec((1, 3 * HH)),
            _full_spec((HH, HH)), _full_spec((1, HH)),
            _full_spec((9, HH)), _full_spec((1, HH)),
        ],
        out_specs=pl.BlockSpec((BS, 4 * HH), lambda i: (i, 0)),
        out_shape=jax.ShapeDtypeStruct((NN, 4 * HH), f32),
    )
    out = tc3(o2.reshape(2, NN, FH), text_f, meta_f,
              attn_Wi, row2(attn_bi), attn_Wo, row2(attn_bo),
              con_W.T, row2(con_b))
    return out


# speedup vs baseline: 3.6096x; 3.6096x over previous
import functools

import jax
import jax.numpy as jnp
from jax import lax
from jax.experimental import pallas as pl
from jax.experimental.pallas import tpu as pltpu
from jax.experimental.pallas import tpu_sc as plsc

NN = 50000
EE = 800000
HH = 48
NR = 4
NH = 4
DH = HH // NH
FH = HH // 2
FP = 32
CH = 128
NCHUNK = EE // CH
BS = 1000
NB = NN // BS
CNT = NR * NN
CNTP = 204800
CB = CNTP // 16

_prec = jax.lax.Precision.HIGHEST


def _dot(a, b):
    return jax.lax.dot_general(a, b, (((a.ndim - 1,), (0,)), ((), ())),
                               precision=_prec, preferred_element_type=jnp.float32)


_sc_mesh = plsc.VectorSubcoreMesh(core_axis_name="c", subcore_axis_name="s")


@functools.partial(
    pl.kernel,
    out_type=jax.ShapeDtypeStruct((2 * CNTP,), jnp.float32),
    mesh=_sc_mesh,
    scratch_types=[
        pltpu.VMEM_SHARED((CNTP,), jnp.float32),
        pltpu.VMEM((CH,), jnp.int32),
        pltpu.VMEM((CH,), jnp.int32),
        pltpu.VMEM((CH,), jnp.int32),
        pltpu.VMEM((CH,), jnp.float32),
    ],
)
def _sc_counts(dst_hbm, typ_hbm, zeros_hbm, cnt_hbm, cnt_sh, dv, tv, cev, ones_v):
    c = lax.axis_index("c")
    s = lax.axis_index("s")
    w = c * 16 + s

    pltpu.sync_copy(zeros_hbm, cnt_sh.at[pl.ds(s * CB, CB)])

    for g in range(CH // 16):
        ones_v[pl.ds(g * 16, 16)] = jnp.full((16,), 1.0, jnp.float32)
    plsc.subcore_barrier()

    def chunk(k, carry):
        cid = w + k * 32
        @pl.when(cid < NCHUNK)
        def _():
            base = cid * CH
            pltpu.sync_copy(dst_hbm.at[pl.ds(base, CH)], dv)
            pltpu.sync_copy(typ_hbm.at[pl.ds(base, CH)], tv)
            for g in range(CH // 16):
                sl = pl.ds(g * 16, 16)
                cev[sl] = tv[sl] * NN + dv[sl]
            pltpu.sync_copy(ones_v, cnt_sh.at[cev], add=True)
        return carry
    lax.fori_loop(0, (NCHUNK + 31) // 32, chunk, 0)
    plsc.subcore_barrier()

    pltpu.sync_copy(cnt_sh.at[pl.ds(s * CB, CB)],
                    cnt_hbm.at[pl.ds(c * CNTP + s * CB, CB)])


@functools.partial(
    pl.kernel,
    out_type=jax.ShapeDtypeStruct((2 * NN, FP), jnp.float32),
    mesh=_sc_mesh,
    scratch_types=[
        pltpu.VMEM_SHARED((NN, FP), jnp.float32),
        pltpu.VMEM((CH,), jnp.int32),
        pltpu.VMEM((CH,), jnp.int32),
        pltpu.VMEM((CH,), jnp.int32),
        pltpu.VMEM((CH,), jnp.int32),
        pltpu.VMEM((CH,), jnp.int32),
        pltpu.VMEM((CH,), jnp.float32),
        pltpu.VMEM((CH, FP), jnp.float32),
        pltpu.SemaphoreType.DMA,
    ],
    compiler_params=pltpu.CompilerParams(use_tc_tiling_on_sc=False),
)
def _sc_edges(src_hbm, dst_hbm, typ_hbm, y_hbm, inv_hbm, dinit_hbm, o_hbm,
              acc, sv, dv, tv, ridx, cev, w_v, rows, sem):
    c = lax.axis_index("c")
    s = lax.axis_index("s")

    def initb(k, carry):
        bid = s + k * 16
        @pl.when(bid < NB)
        def _():
            pltpu.sync_copy(dinit_hbm.at[pl.ds(c * NN + bid * BS, BS)],
                            acc.at[pl.ds(bid * BS, BS)])
        return carry
    lax.fori_loop(0, (NB + 15) // 16, initb, 0)
    plsc.subcore_barrier()

    def chunk(k, carry):
        cid = s + k * 16
        @pl.when(cid < NCHUNK)
        def _():
            base = cid * CH
            pltpu.sync_copy(src_hbm.at[pl.ds(base, CH)], sv)
            pltpu.sync_copy(dst_hbm.at[pl.ds(base, CH)], dv)
            pltpu.sync_copy(typ_hbm.at[pl.ds(base, CH)], tv)
            off = c * (NR * NN)
            for g in range(CH // 16):
                sl = pl.ds(g * 16, 16)
                t16 = tv[sl]
                ridx[sl] = off + t16 * NN + sv[sl]
                cev[sl] = t16 * NN + dv[sl]
            pltpu.async_copy(inv_hbm.at[cev], w_v, sem).wait()
            pltpu.async_copy(y_hbm.at[ridx], rows, sem).wait()
            for g in range(CH // 16):
                wg = w_v[pl.ds(g * 16, 16)]
                for l in range(16):
                    i = g * 16 + l
                    wb = jnp.broadcast_to(wg[l], (16,))
                    for h in range(0, FP, 16):
                        rows[i, pl.ds(h, 16)] = rows[i, pl.ds(h, 16)] * wb
            pltpu.sync_copy(rows, acc.at[dv], add=True)
        return carry
    lax.fori_loop(0, (NCHUNK + 15) // 16, chunk, 0)
    plsc.subcore_barrier()

    def outb(k, carry):
        bid = s + k * 16
        @pl.when(bid < NB)
        def _():
            pltpu.sync_copy(acc.at[pl.ds(bid * BS, BS)],
                            o_hbm.at[pl.ds(c * NN + bid * BS, BS)])
        return carry
    lax.fori_loop(0, (NB + 15) // 16, outb, 0)


def _tc1_body(meta_ref, text_ref, mW1, mb1, mW2, mb2, tW1, tb1, tW2, tb2,
              g1W, g1root, g1b, cnt_ref,
              metaf_ref, textf_ref, y_ref, d_ref, inv_ref):
    mf = jnp.maximum(_dot(meta_ref[...], mW1[...].T) + mb1[...], 0.0)
    mf = jnp.maximum(_dot(mf, mW2[...].T) + mb2[...], 0.0)
    tf = jnp.maximum(_dot(text_ref[...], tW1[...].T) + tb1[...], 0.0)
    tf = jnp.maximum(_dot(tf, tW2[...].T) + tb2[...], 0.0)
    metaf_ref[...] = mf
    textf_ref[...] = tf
    for r in range(NR):
        yr = _dot(mf, g1W[r, :HH, :]) + _dot(tf, g1W[r, HH:, :])
        y_ref[0, r] = yr[:, :FP]
        y_ref[1, r] = yr[:, FP:]
    d = _dot(mf, g1root[:HH, :]) + _dot(tf, g1root[HH:, :]) + g1b[...]
    d_ref[0] = d[:, :FP]
    d_ref[1] = d[:, FP:]
    csum = cnt_ref[0, 0] + cnt_ref[1, 0]
    inv_ref[0] = 1.0 / jnp.maximum(csum, 1.0)


def _tc2_body(o1_ref, g2W, g2root, g2b, y_ref, d_ref):
    ra = jnp.maximum(o1_ref[0], 0.0)
    rb = jnp.maximum(o1_ref[1], 0.0)
    for r in range(NR):
        yr = _dot(ra, g2W[r, :FP, :]) + _dot(rb, g2W[r, FP:, :])
        y_ref[0, r] = yr[:, :FP]
        y_ref[1, r] = yr[:, FP:]
    d = _dot(ra, g2root[:FP, :]) + _dot(rb, g2root[FP:, :]) + g2b[...]
    d_ref[0] = d[:, :FP]
    d_ref[1] = d[:, FP:]


def _tc3_body(o2_ref, textf_ref, metaf_ref, Wip, WiT, bi, Wo, bo, conWT, con_b,
              out_ref):
    ga = jnp.maximum(o2_ref[0], 0.0)
    gb = jnp.maximum(o2_ref[1], 0.0)
    tf = textf_ref[...]
    mf = metaf_ref[...]

    def proj(p):
        Mp = Wip[p]
        M = WiT[p]
        b = bi[...][:, p * HH:(p + 1) * HH]
        qg = _dot(ga, Mp[:FP, :]) + _dot(gb, Mp[FP:, :]) + b
        qt = _dot(tf, M) + b
        qm = _dot(mf, M) + b
        return (qg, qt, qm)

    q = proj(0)
    kk = proj(1)
    v = proj(2)

    hsel = (jax.lax.broadcasted_iota(jnp.int32, (HH, NH), 0) // DH ==
            jax.lax.broadcasted_iota(jnp.int32, (HH, NH), 1)).astype(jnp.float32)
    scale = 1.0 / (DH ** 0.5)

    s = [[_dot(q[i] * kk[j], hsel) * scale for j in range(3)] for i in range(3)]
    a = []
    aw = []
    for i in range(3):
        m = jnp.maximum(jnp.maximum(s[i][0], s[i][1]), s[i][2])
        e = [jnp.exp(s[i][j] - m) for j in range(3)]
        z = e[0] + e[1] + e[2]
        ai = [e[j] / z for j in range(3)]
        a.append(ai)
        aw.append([jnp.sum(ai[j], axis=1, keepdims=True) * (1.0 / NH)
                   for j in range(3)])

    hselT = hsel.T
    WoT = Wo[...].T
    f_out = []
    for i in range(3):
        oi = (_dot(a[i][0], hselT) * v[0] +
              _dot(a[i][1], hselT) * v[1] +
              _dot(a[i][2], hselT) * v[2])
        f_out.append(_dot(oi, WoT) + bo[...])

    fc = con_b[...]
    for i in range(3):
        for j in range(3):
            fc = fc + aw[i][j] * conWT[3 * i + j:3 * i + j + 1, :]

    out_ref[:, 0:HH] = f_out[0]
    out_ref[:, HH:2 * HH] = f_out[1]
    out_ref[:, 2 * HH:3 * HH] = f_out[2]
    out_ref[:, 3 * HH:4 * HH] = fc


def _full_spec(shape):
    return pl.BlockSpec(shape, lambda *args: tuple(0 for _ in shape))


def _pad48(w, axis):
    a = lax.slice_in_dim(w, 0, FH, axis=axis)
    b = lax.slice_in_dim(w, FH, HH, axis=axis)
    zshape = list(w.shape)
    zshape[axis] = FP - FH
    z = jnp.zeros(zshape, w.dtype)
    return jnp.concatenate([a, z, b, z], axis=axis)


def kernel(meta, text, edge_index, edge_type, meta_W1, meta_b1, meta_W2, meta_b2,
           text_W1, text_b1, text_W2, text_b2, g1_W, g1_root, g1_b,
           g2_W, g2_root, g2_b, attn_Wi, attn_bi, attn_Wo, attn_bo, con_W, con_b):
    f32 = jnp.float32
    src = edge_index[0]
    dst = edge_index[1]
    typ = edge_type
    zeros_cb = jnp.zeros((CB,), f32)

    cnt = _sc_counts(dst, typ, zeros_cb)

    row2 = lambda x: x.reshape(1, -1)
    tc1 = pl.pallas_call(
        _tc1_body,
        grid=(NB,),
        in_specs=[
            pl.BlockSpec((BS, 16), lambda i: (i, 0)),
            pl.BlockSpec((BS, 768), lambda i: (i, 0)),
            _full_spec((HH, 16)), _full_spec((1, HH)),
            _full_spec((HH, HH)), _full_spec((1, HH)),
            _full_spec((HH, 768)), _full_spec((1, HH)),
            _full_spec((HH, HH)), _full_spec((1, HH)),
            _full_spec((NR, 2 * HH, 2 * FP)), _full_spec((2 * HH, 2 * FP)),
            _full_spec((1, 2 * FP)),
            pl.BlockSpec((2, 1, 1, CNTP // NB), lambda i: (0, i, 0, 0)),
        ],
        out_specs=[
            pl.BlockSpec((BS, HH), lambda i: (i, 0)),
            pl.BlockSpec((BS, HH), lambda i: (i, 0)),
            pl.BlockSpec((2, NR, BS, FP), lambda i: (0, 0, i, 0)),
            pl.BlockSpec((2, BS, FP), lambda i: (0, i, 0)),
            pl.BlockSpec((1, 1, CNTP // NB), lambda i: (i, 0, 0)),
        ],
        out_shape=[
            jax.ShapeDtypeStruct((NN, HH), f32),
            jax.ShapeDtypeStruct((NN, HH), f32),
            jax.ShapeDtypeStruct((2, NR, NN, FP), f32),
            jax.ShapeDtypeStruct((2, NN, FP), f32),
            jax.ShapeDtypeStruct((NB, 1, CNTP // NB), f32),
        ],
    )
    meta_f, text_f, y1, d1, inv = tc1(
        meta, text, meta_W1, row2(meta_b1), meta_W2, row2(meta_b2),
        text_W1, row2(text_b1), text_W2, row2(text_b2),
        _pad48(g1_W, 2), _pad48(g1_root, 1), _pad48(row2(g1_b), 1),
        cnt.reshape(2, NB, 1, CNTP // NB))

    inv_flat = inv.reshape(CNTP)
    o1 = _sc_edges(src, dst, typ, y1.reshape(2 * NR * NN, FP), inv_flat,
                   d1.reshape(2 * NN, FP))

    tc2 = pl.pallas_call(
        _tc2_body,
        grid=(NB,),
        in_specs=[
            pl.BlockSpec((2, BS, FP), lambda i: (0, i, 0)),
            _full_spec((NR, 2 * FP, 2 * FP)), _full_spec((2 * FP, 2 * FP)),
            _full_spec((1, 2 * FP)),
        ],
        out_specs=[
            pl.BlockSpec((2, NR, BS, FP), lambda i: (0, 0, i, 0)),
            pl.BlockSpec((2, BS, FP), lambda i: (0, i, 0)),
        ],
        out_shape=[
            jax.ShapeDtypeStruct((2, NR, NN, FP), f32),
            jax.ShapeDtypeStruct((2, NN, FP), f32),
        ],
    )
    y2, d2 = tc2(o1.reshape(2, NN, FP),
                 _pad48(_pad48(g2_W, 1), 2),
                 _pad48(_pad48(g2_root, 0), 1),
                 _pad48(row2(g2_b), 1))

    o2 = _sc_edges(src, dst, typ, y2.reshape(2 * NR * NN, FP), inv_flat,
                   d2.reshape(2 * NN, FP))

    WiT = jnp.stack([attn_Wi[p * HH:(p + 1) * HH].T for p in range(3)])
    Wip = _pad48(WiT, 1)
    tc3 = pl.pallas_call(
        _tc3_body,
        grid=(NB,),
        in_specs=[
            pl.BlockSpec((2, BS, FP), lambda i: (0, i, 0)),
            pl.BlockSpec((BS, HH), lambda i: (i, 0)),
            pl.BlockSpec((BS, HH), lambda i: (i, 0)),
            _full_spec((3, 2 * FP, HH)), _full_spec((3, HH, HH)),
            _full_spec((1, 3 * HH)),
            _full_spec((HH, HH)), _full_spec((1, HH)),
            _full_spec((9, HH)), _full_spec((1, HH)),
        ],
        out_specs=pl.BlockSpec((BS, 4 * HH), lambda i: (i, 0)),
        out_shape=jax.ShapeDtypeStruct((NN, 4 * HH), f32),
    )
    out = tc3(o2.reshape(2, NN, FP), text_f, meta_f,
              Wip, WiT, row2(attn_bi), attn_Wo, row2(attn_bo),
              con_W.T, row2(con_b))
    return out


# speedup vs baseline: 4.8897x; 1.3546x over previous
import functools

import jax
import jax.numpy as jnp
from jax import lax
from jax.experimental import pallas as pl
from jax.experimental.pallas import tpu as pltpu
from jax.experimental.pallas import tpu_sc as plsc

NN = 50000
EE = 800000
HH = 48
NR = 4
NH = 4
DH = HH // NH
FH = HH // 2
FP = 32
CH = 128
EP = 819200
NCHUNK = EP // CH
CPT = NCHUNK // 16
BLK = 40
BE = BLK * CH
BS = 1000
NB = NN // BS
CNT = NR * NN
CNTP = 204800
CB = CNTP // 16

_prec = jax.lax.Precision.HIGHEST


def _dot(a, b):
    return jax.lax.dot_general(a, b, (((a.ndim - 1,), (0,)), ((), ())),
                               precision=_prec, preferred_element_type=jnp.float32)


_sc_mesh = plsc.VectorSubcoreMesh(core_axis_name="c", subcore_axis_name="s")


@functools.partial(
    pl.kernel,
    out_type=jax.ShapeDtypeStruct((2 * CNTP,), jnp.float32),
    mesh=_sc_mesh,
    scratch_types=[
        pltpu.VMEM_SHARED((CNTP,), jnp.float32),
        pltpu.VMEM((BE,), jnp.int32),
        pltpu.VMEM((BE,), jnp.int32),
        pltpu.VMEM((CH,), jnp.int32),
        pltpu.VMEM((CH,), jnp.float32),
    ],
)
def _sc_counts(dst_hbm, typ_hbm, zeros_hbm, cnt_hbm, cnt_sh, dve, tve, cev, val_v):
    c = lax.axis_index("c")
    s = lax.axis_index("s")
    w = c * 16 + s

    pltpu.sync_copy(zeros_hbm, cnt_sh.at[pl.ds(s * CB, CB)])
    plsc.subcore_barrier()

    iota = lax.iota(jnp.int32, 16)
    cpt = NCHUNK // 32

    def block(b, carry):
        ebase = w * (cpt * CH) + b * BE
        pltpu.sync_copy(dst_hbm.at[pl.ds(ebase, BE)], dve)
        pltpu.sync_copy(typ_hbm.at[pl.ds(ebase, BE)], tve)

        def chunk(j, carry2):
            off = j * CH
            for g in range(CH // 16):
                sl = pl.ds(off + g * 16, 16)
                gl = pl.ds(g * 16, 16)
                cev[gl] = tve[sl] * NN + dve[sl]
                ge = jnp.broadcast_to(ebase + off + g * 16, (16,)) + iota
                val_v[gl] = jnp.where(ge < EE, 1.0, 0.0)
            pltpu.sync_copy(val_v, cnt_sh.at[cev], add=True)
            return carry2
        lax.fori_loop(0, BLK, chunk, 0)
        return carry
    lax.fori_loop(0, cpt // BLK, block, 0)
    plsc.subcore_barrier()

    pltpu.sync_copy(cnt_sh.at[pl.ds(s * CB, CB)],
                    cnt_hbm.at[pl.ds(c * CNTP + s * CB, CB)])


def _make_sc_edges(compute_w):
    if compute_w:
        out_type = (jax.ShapeDtypeStruct((2 * NN, FP), jnp.float32),
                    jax.ShapeDtypeStruct((2 * EP,), jnp.float32))
    else:
        out_type = jax.ShapeDtypeStruct((2 * NN, FP), jnp.float32)

    @functools.partial(
        pl.kernel,
        out_type=out_type,
        mesh=_sc_mesh,
        scratch_types=[
            pltpu.VMEM_SHARED((NN, FP), jnp.float32),
            pltpu.VMEM((BE,), jnp.int32),
            pltpu.VMEM((BE,), jnp.int32),
            pltpu.VMEM((BE,), jnp.int32),
            pltpu.VMEM((BE,), jnp.float32),
            pltpu.VMEM((CH,), jnp.int32),
            pltpu.VMEM((CH,), jnp.int32),
            pltpu.VMEM((CH,), jnp.int32),
            pltpu.VMEM((CH,), jnp.int32),
            pltpu.VMEM((CH,), jnp.int32),
            pltpu.VMEM((CH,), jnp.int32),
            pltpu.VMEM((CH, FP), jnp.float32),
            pltpu.VMEM((CH, FP), jnp.float32),
            pltpu.SemaphoreType.DMA,
            pltpu.SemaphoreType.DMA,
            pltpu.SemaphoreType.DMA,
            pltpu.SemaphoreType.DMA,
            pltpu.SemaphoreType.DMA,
        ],
        compiler_params=pltpu.CompilerParams(use_tc_tiling_on_sc=False),
    )
    def _sc_edges(src_hbm, dst_hbm, typ_hbm, y_hbm, winv_hbm, dinit_hbm,
                  o_hbm, *rest):
        if compute_w:
            (w_hbm, acc, sve, dve, tve, wve, ridx0, ridx1, dv0, dv1,
             cev0, cev1, rows0, rows1, sg0, sg1, sw0, sw1, ss) = rest
        else:
            (acc, sve, dve, tve, wve, ridx0, ridx1, dv0, dv1,
             cev0, cev1, rows0, rows1, sg0, sg1, sw0, sw1, ss) = rest
            w_hbm = None
        c = lax.axis_index("c")
        s = lax.axis_index("s")
        coff = c * (NR * NN)
        iota = lax.iota(jnp.int32, 16)

        def initb(k, carry):
            bid = s + k * 16
            @pl.when(bid < NB)
            def _():
                pltpu.sync_copy(dinit_hbm.at[pl.ds(c * NN + bid * BS, BS)],
                                acc.at[pl.ds(bid * BS, BS)])
            return carry
        lax.fori_loop(0, (NB + 15) // 16, initb, 0)
        plsc.subcore_barrier()

        def block(b, carry):
            ebase = s * (CPT * CH) + b * BE
            pltpu.sync_copy(src_hbm.at[pl.ds(ebase, BE)], sve)
            pltpu.sync_copy(dst_hbm.at[pl.ds(ebase, BE)], dve)
            pltpu.sync_copy(typ_hbm.at[pl.ds(ebase, BE)], tve)
            if not compute_w:
                pltpu.sync_copy(winv_hbm.at[pl.ds(c * EP + ebase, BE)], wve)

            def do_idx(jj, ridx, dvb, cev):
                off = jj * CH
                for g in range(CH // 16):
                    sl = pl.ds(off + g * 16, 16)
                    gl = pl.ds(g * 16, 16)
                    t16 = tve[sl]
                    ridx[gl] = coff + t16 * NN + sve[sl]
                    dvb[gl] = dve[sl]
                    if compute_w:
                        cev[gl] = t16 * NN + dve[sl]

            def mask_w(jj):
                off = jj * CH
                for g in range(CH // 16):
                    sl = pl.ds(off + g * 16, 16)
                    ge = jnp.broadcast_to(ebase + off + g * 16, (16,)) + iota
                    wve[sl] = jnp.where(ge < EE, wve[sl], 0.0)

            def scale(rows, jj):
                off = jj * CH
                for g in range(CH // 16):
                    wg = wve[pl.ds(off + g * 16, 16)]
                    for l in range(16):
                        i = g * 16 + l
                        wb = jnp.broadcast_to(wg[l], (16,))
                        for h in range(0, FP, 16):
                            rows[i, pl.ds(h, 16)] = rows[i, pl.ds(h, 16)] * wb

            def pair(j2, carry2):
                jj0 = j2 * 2
                jj1 = jj0 + 1
                do_idx(jj0, ridx0, dv0, cev0)
                g0 = pltpu.async_copy(y_hbm.at[ridx0], rows0, sg0)
                if compute_w:
                    w0 = pltpu.async_copy(winv_hbm.at[cev0],
                                          wve.at[pl.ds(jj0 * CH, CH)], sw0)
                do_idx(jj1, ridx1, dv1, cev1)
                g1 = pltpu.async_copy(y_hbm.at[ridx1], rows1, sg1)
                if compute_w:
                    w1 = pltpu.async_copy(winv_hbm.at[cev1],
                                          wve.at[pl.ds(jj1 * CH, CH)], sw1)
                    w0.wait()
                    mask_w(jj0)
                g0.wait()
                scale(rows0, jj0)
                s0 = pltpu.async_copy(rows0, acc.at[dv0], ss, add=True)
                if compute_w:
                    w1.wait()
                    mask_w(jj1)
                g1.wait()
                scale(rows1, jj1)
                s1 = pltpu.async_copy(rows1, acc.at[dv1], ss, add=True)
                s0.wait()
                s1.wait()
                return carry2
            lax.fori_loop(0, BLK // 2, pair, 0)
            if compute_w:
                pltpu.sync_copy(wve, w_hbm.at[pl.ds(c * EP + ebase, BE)])
            return carry
        lax.fori_loop(0, CPT // BLK, block, 0)
        plsc.subcore_barrier()

        def outb(k, carry):
            bid = s + k * 16
            @pl.when(bid < NB)
            def _():
                pltpu.sync_copy(acc.at[pl.ds(bid * BS, BS)],
                                o_hbm.at[pl.ds(c * NN + bid * BS, BS)])
            return carry
        lax.fori_loop(0, (NB + 15) // 16, outb, 0)

    return _sc_edges


_sc_edges_l1 = _make_sc_edges(True)
_sc_edges_l2 = _make_sc_edges(False)


def _tc1_body(meta_ref, text_ref, mW1, mb1, mW2, mb2, tW1, tb1, tW2, tb2,
              g1W, g1root, g1b, cnt_ref,
              metaf_ref, textf_ref, y_ref, d_ref, inv_ref):
    mf = jnp.maximum(_dot(meta_ref[...], mW1[...].T) + mb1[...], 0.0)
    mf = jnp.maximum(_dot(mf, mW2[...].T) + mb2[...], 0.0)
    tf = jnp.maximum(_dot(text_ref[...], tW1[...].T) + tb1[...], 0.0)
    tf = jnp.maximum(_dot(tf, tW2[...].T) + tb2[...], 0.0)
    metaf_ref[...] = mf
    textf_ref[...] = tf
    for r in range(NR):
        yr = _dot(mf, g1W[r, :HH, :]) + _dot(tf, g1W[r, HH:, :])
        y_ref[0, r] = yr[:, :FP]
        y_ref[1, r] = yr[:, FP:]
    d = _dot(mf, g1root[:HH, :]) + _dot(tf, g1root[HH:, :]) + g1b[...]
    d_ref[0] = d[:, :FP]
    d_ref[1] = d[:, FP:]
    csum = cnt_ref[0, 0] + cnt_ref[1, 0]
    inv_ref[0] = 1.0 / jnp.maximum(csum, 1.0)


def _tc2_body(o1_ref, g2W, g2root, g2b, y_ref, d_ref):
    ra = jnp.maximum(o1_ref[0], 0.0)
    rb = jnp.maximum(o1_ref[1], 0.0)
    for r in range(NR):
        yr = _dot(ra, g2W[r, :FP, :]) + _dot(rb, g2W[r, FP:, :])
        y_ref[0, r] = yr[:, :FP]
        y_ref[1, r] = yr[:, FP:]
    d = _dot(ra, g2root[:FP, :]) + _dot(rb, g2root[FP:, :]) + g2b[...]
    d_ref[0] = d[:, :FP]
    d_ref[1] = d[:, FP:]


def _tc3_body(o2_ref, textf_ref, metaf_ref, Wip, WiT, bi, Wo, bo, conWT, con_b,
              out_ref):
    ga = jnp.maximum(o2_ref[0], 0.0)
    gb = jnp.maximum(o2_ref[1], 0.0)
    tf = textf_ref[...]
    mf = metaf_ref[...]

    def proj(p):
        Mp = Wip[p]
        M = WiT[p]
        b = bi[...][:, p * HH:(p + 1) * HH]
        qg = _dot(ga, Mp[:FP, :]) + _dot(gb, Mp[FP:, :]) + b
        qt = _dot(tf, M) + b
        qm = _dot(mf, M) + b
        return (qg, qt, qm)

    q = proj(0)
    kk = proj(1)
    v = proj(2)

    hsel = (jax.lax.broadcasted_iota(jnp.int32, (HH, NH), 0) // DH ==
            jax.lax.broadcasted_iota(jnp.int32, (HH, NH), 1)).astype(jnp.float32)
    scale = 1.0 / (DH ** 0.5)

    s = [[_dot(q[i] * kk[j], hsel) * scale for j in range(3)] for i in range(3)]
    a = []
    aw = []
    for i in range(3):
        m = jnp.maximum(jnp.maximum(s[i][0], s[i][1]), s[i][2])
        e = [jnp.exp(s[i][j] - m) for j in range(3)]
        z = e[0] + e[1] + e[2]
        ai = [e[j] / z for j in range(3)]
        a.append(ai)
        aw.append([jnp.sum(ai[j], axis=1, keepdims=True) * (1.0 / NH)
                   for j in range(3)])

    hselT = hsel.T
    WoT = Wo[...].T
    f_out = []
    for i in range(3):
        oi = (_dot(a[i][0], hselT) * v[0] +
              _dot(a[i][1], hselT) * v[1] +
              _dot(a[i][2], hselT) * v[2])
        f_out.append(_dot(oi, WoT) + bo[...])

    fc = con_b[...]
    for i in range(3):
        for j in range(3):
            fc = fc + aw[i][j] * conWT[3 * i + j:3 * i + j + 1, :]

    out_ref[:, 0:HH] = f_out[0]
    out_ref[:, HH:2 * HH] = f_out[1]
    out_ref[:, 2 * HH:3 * HH] = f_out[2]
    out_ref[:, 3 * HH:4 * HH] = fc


def _full_spec(shape):
    return pl.BlockSpec(shape, lambda *args: tuple(0 for _ in shape))


def _pad48(w, axis):
    a = lax.slice_in_dim(w, 0, FH, axis=axis)
    b = lax.slice_in_dim(w, FH, HH, axis=axis)
    zshape = list(w.shape)
    zshape[axis] = FP - FH
    z = jnp.zeros(zshape, w.dtype)
    return jnp.concatenate([a, z, b, z], axis=axis)


def kernel(meta, text, edge_index, edge_type, meta_W1, meta_b1, meta_W2, meta_b2,
           text_W1, text_b1, text_W2, text_b2, g1_W, g1_root, g1_b,
           g2_W, g2_root, g2_b, attn_Wi, attn_bi, attn_Wo, attn_bo, con_W, con_b):
    f32 = jnp.float32
    ei = jnp.pad(edge_index, ((0, 0), (0, EP - EE)))
    src = ei[0]
    dst = ei[1]
    typ = jnp.pad(edge_type, (0, EP - EE))
    zeros_cb = jnp.zeros((CB,), f32)

    cnt = _sc_counts(dst, typ, zeros_cb)

    row2 = lambda x: x.reshape(1, -1)
    tc1 = pl.pallas_call(
        _tc1_body,
        grid=(NB,),
        in_specs=[
            pl.BlockSpec((BS, 16), lambda i: (i, 0)),
            pl.BlockSpec((BS, 768), lambda i: (i, 0)),
            _full_spec((HH, 16)), _full_spec((1, HH)),
            _full_spec((HH, HH)), _full_spec((1, HH)),
            _full_spec((HH, 768)), _full_spec((1, HH)),
            _full_spec((HH, HH)), _full_spec((1, HH)),
            _full_spec((NR, 2 * HH, 2 * FP)), _full_spec((2 * HH, 2 * FP)),
            _full_spec((1, 2 * FP)),
            pl.BlockSpec((2, 1, 1, CNTP // NB), lambda i: (0, i, 0, 0)),
        ],
        out_specs=[
            pl.BlockSpec((BS, HH), lambda i: (i, 0)),
            pl.BlockSpec((BS, HH), lambda i: (i, 0)),
            pl.BlockSpec((2, NR, BS, FP), lambda i: (0, 0, i, 0)),
            pl.BlockSpec((2, BS, FP), lambda i: (0, i, 0)),
            pl.BlockSpec((1, 1, CNTP // NB), lambda i: (i, 0, 0)),
        ],
        out_shape=[
            jax.ShapeDtypeStruct((NN, HH), f32),
            jax.ShapeDtypeStruct((NN, HH), f32),
            jax.ShapeDtypeStruct((2, NR, NN, FP), f32),
            jax.ShapeDtypeStruct((2, NN, FP), f32),
            jax.ShapeDtypeStruct((NB, 1, CNTP // NB), f32),
        ],
    )
    meta_f, text_f, y1, d1, inv = tc1(
        meta, text, meta_W1, row2(meta_b1), meta_W2, row2(meta_b2),
        text_W1, row2(text_b1), text_W2, row2(text_b2),
        _pad48(g1_W, 2), _pad48(g1_root, 1), _pad48(row2(g1_b), 1),
        cnt.reshape(2, NB, 1, CNTP // NB))

    inv_flat = inv.reshape(CNTP)
    o1, w_edges = _sc_edges_l1(src, dst, typ, y1.reshape(2 * NR * NN, FP),
                               inv_flat, d1.reshape(2 * NN, FP))

    tc2 = pl.pallas_call(
        _tc2_body,
        grid=(NB,),
        in_specs=[
            pl.BlockSpec((2, BS, FP), lambda i: (0, i, 0)),
            _full_spec((NR, 2 * FP, 2 * FP)), _full_spec((2 * FP, 2 * FP)),
            _full_spec((1, 2 * FP)),
        ],
        out_specs=[
            pl.BlockSpec((2, NR, BS, FP), lambda i: (0, 0, i, 0)),
            pl.BlockSpec((2, BS, FP), lambda i: (0, i, 0)),
        ],
        out_shape=[
            jax.ShapeDtypeStruct((2, NR, NN, FP), f32),
            jax.ShapeDtypeStruct((2, NN, FP), f32),
        ],
    )
    y2, d2 = tc2(o1.reshape(2, NN, FP),
                 _pad48(_pad48(g2_W, 1), 2),
                 _pad48(_pad48(g2_root, 0), 1),
                 _pad48(row2(g2_b), 1))

    o2 = _sc_edges_l2(src, dst, typ, y2.reshape(2 * NR * NN, FP), w_edges,
                      d2.reshape(2 * NN, FP))

    WiT = jnp.stack([attn_Wi[p * HH:(p + 1) * HH].T for p in range(3)])
    Wip = _pad48(WiT, 1)
    tc3 = pl.pallas_call(
        _tc3_body,
        grid=(NB,),
        in_specs=[
            pl.BlockSpec((2, BS, FP), lambda i: (0, i, 0)),
            pl.BlockSpec((BS, HH), lambda i: (i, 0)),
            pl.BlockSpec((BS, HH), lambda i: (i, 0)),
            _full_spec((3, 2 * FP, HH)), _full_spec((3, HH, HH)),
            _full_spec((1, 3 * HH)),
            _full_spec((HH, HH)), _full_spec((1, HH)),
            _full_spec((9, HH)), _full_spec((1, HH)),
        ],
        out_specs=pl.BlockSpec((BS, 4 * HH), lambda i: (i, 0)),
        out_shape=jax.ShapeDtypeStruct((NN, 4 * HH), f32),
    )
    out = tc3(o2.reshape(2, NN, FP), text_f, meta_f,
              Wip, WiT, row2(attn_bi), attn_Wo, row2(attn_bo),
              con_W.T, row2(con_b))
    return out


# speedup vs baseline: 9.0909x; 1.8592x over previous
import functools

import jax
import jax.numpy as jnp
from jax import lax
from jax.experimental import pallas as pl
from jax.experimental.pallas import tpu as pltpu
from jax.experimental.pallas import tpu_sc as plsc

NN = 50000
EE = 800000
HH = 48
NR = 4
NH = 4
DH = HH // NH
FH = HH // 2
FP = 32
CH = 128
EP = 819200
NCHUNK = EP // CH
CPT = NCHUNK // 16
BLK = 40
BE = BLK * CH
BS = 2000
NB = NN // BS
NBS = 1000
NBB = NN // NBS
CNT = NR * NN
CNTP = 204800
CB = CNTP // 16

_prec = jax.lax.Precision.DEFAULT


def _dot(a, b):
    return jax.lax.dot_general(a, b, (((a.ndim - 1,), (0,)), ((), ())),
                               precision=_prec, preferred_element_type=jnp.float32)


_sc_mesh = plsc.VectorSubcoreMesh(core_axis_name="c", subcore_axis_name="s")


@functools.partial(
    pl.kernel,
    out_type=jax.ShapeDtypeStruct((2 * CNTP,), jnp.float32),
    mesh=_sc_mesh,
    scratch_types=[
        pltpu.VMEM_SHARED((CNTP,), jnp.float32),
        pltpu.VMEM((BE,), jnp.int32),
        pltpu.VMEM((BE,), jnp.int32),
        pltpu.VMEM((CH,), jnp.int32),
        pltpu.VMEM((CH,), jnp.float32),
    ],
)
def _sc_counts(dst_hbm, typ_hbm, zeros_hbm, cnt_hbm, cnt_sh, dve, tve, cev, val_v):
    c = lax.axis_index("c")
    s = lax.axis_index("s")
    w = c * 16 + s

    pltpu.sync_copy(zeros_hbm, cnt_sh.at[pl.ds(s * CB, CB)])
    plsc.subcore_barrier()

    iota = lax.iota(jnp.int32, 16)
    cpt = NCHUNK // 32

    def block(b, carry):
        ebase = w * (cpt * CH) + b * BE
        pltpu.sync_copy(dst_hbm.at[pl.ds(ebase, BE)], dve)
        pltpu.sync_copy(typ_hbm.at[pl.ds(ebase, BE)], tve)

        def chunk(j, carry2):
            off = j * CH
            for g in range(CH // 16):
                sl = pl.ds(off + g * 16, 16)
                gl = pl.ds(g * 16, 16)
                cev[gl] = tve[sl] * NN + dve[sl]
                ge = jnp.broadcast_to(ebase + off + g * 16, (16,)) + iota
                val_v[gl] = jnp.where(ge < EE, 1.0, 0.0)
            pltpu.sync_copy(val_v, cnt_sh.at[cev], add=True)
            return carry2
        lax.fori_loop(0, BLK, chunk, 0)
        return carry
    lax.fori_loop(0, cpt // BLK, block, 0)
    plsc.subcore_barrier()

    pltpu.sync_copy(cnt_sh.at[pl.ds(s * CB, CB)],
                    cnt_hbm.at[pl.ds(c * CNTP + s * CB, CB)])


def _make_sc_edges(compute_w):
    if compute_w:
        out_type = (jax.ShapeDtypeStruct((2 * NN, FP), jnp.float32),
                    jax.ShapeDtypeStruct((2 * EP,), jnp.float32))
    else:
        out_type = jax.ShapeDtypeStruct((2 * NN, FP), jnp.float32)

    @functools.partial(
        pl.kernel,
        out_type=out_type,
        mesh=_sc_mesh,
        scratch_types=[
            pltpu.VMEM_SHARED((NN, FP), jnp.float32),
            pltpu.VMEM((BE,), jnp.int32),
            pltpu.VMEM((BE,), jnp.int32),
            pltpu.VMEM((BE,), jnp.int32),
            pltpu.VMEM((BE,), jnp.float32),
            pltpu.VMEM((CH,), jnp.int32),
            pltpu.VMEM((CH,), jnp.int32),
            pltpu.VMEM((CH,), jnp.int32),
            pltpu.VMEM((CH,), jnp.int32),
            pltpu.VMEM((CH,), jnp.int32),
            pltpu.VMEM((CH,), jnp.int32),
            pltpu.VMEM((CH, FP), jnp.float32),
            pltpu.VMEM((CH, FP), jnp.float32),
            pltpu.SemaphoreType.DMA,
            pltpu.SemaphoreType.DMA,
            pltpu.SemaphoreType.DMA,
            pltpu.SemaphoreType.DMA,
            pltpu.SemaphoreType.DMA,
        ],
        compiler_params=pltpu.CompilerParams(use_tc_tiling_on_sc=False),
    )
    def _sc_edges(src_hbm, dst_hbm, typ_hbm, y_hbm, winv_hbm, dinit_hbm,
                  o_hbm, *rest):
        if compute_w:
            (w_hbm, acc, sve, dve, tve, wve, ridx0, ridx1, dv0, dv1,
             cev0, cev1, rows0, rows1, sg0, sg1, sw0, sw1, ss) = rest
        else:
            (acc, sve, dve, tve, wve, ridx0, ridx1, dv0, dv1,
             cev0, cev1, rows0, rows1, sg0, sg1, sw0, sw1, ss) = rest
            w_hbm = None
        c = lax.axis_index("c")
        s = lax.axis_index("s")
        coff = c * (NR * NN)
        iota = lax.iota(jnp.int32, 16)

        def initb(k, carry):
            bid = s + k * 16
            @pl.when(bid < NBB)
            def _():
                pltpu.sync_copy(dinit_hbm.at[pl.ds(c * NN + bid * NBS, NBS)],
                                acc.at[pl.ds(bid * NBS, NBS)])
            return carry
        lax.fori_loop(0, (NBB + 15) // 16, initb, 0)
        plsc.subcore_barrier()

        def block(b, carry):
            ebase = s * (CPT * CH) + b * BE
            pltpu.sync_copy(src_hbm.at[pl.ds(ebase, BE)], sve)
            pltpu.sync_copy(dst_hbm.at[pl.ds(ebase, BE)], dve)
            pltpu.sync_copy(typ_hbm.at[pl.ds(ebase, BE)], tve)
            if not compute_w:
                pltpu.sync_copy(winv_hbm.at[pl.ds(c * EP + ebase, BE)], wve)

            def do_idx(jj, ridx, dvb, cev):
                off = jj * CH
                for g in range(CH // 16):
                    sl = pl.ds(off + g * 16, 16)
                    gl = pl.ds(g * 16, 16)
                    t16 = tve[sl]
                    ridx[gl] = coff + t16 * NN + sve[sl]
                    dvb[gl] = dve[sl]
                    if compute_w:
                        cev[gl] = t16 * NN + dve[sl]

            def mask_w(jj):
                off = jj * CH
                for g in range(CH // 16):
                    sl = pl.ds(off + g * 16, 16)
                    ge = jnp.broadcast_to(ebase + off + g * 16, (16,)) + iota
                    wve[sl] = jnp.where(ge < EE, wve[sl], 0.0)

            def scale(rows, jj):
                off = jj * CH
                for g in range(CH // 16):
                    wg = wve[pl.ds(off + g * 16, 16)]
                    for l in range(16):
                        i = g * 16 + l
                        wb = jnp.broadcast_to(wg[l], (16,))
                        for h in range(0, FP, 16):
                            rows[i, pl.ds(h, 16)] = rows[i, pl.ds(h, 16)] * wb

            def pair(j2, carry2):
                jj0 = j2 * 2
                jj1 = jj0 + 1
                do_idx(jj0, ridx0, dv0, cev0)
                g0 = pltpu.async_copy(y_hbm.at[ridx0], rows0, sg0)
                if compute_w:
                    w0 = pltpu.async_copy(winv_hbm.at[cev0],
                                          wve.at[pl.ds(jj0 * CH, CH)], sw0)
                do_idx(jj1, ridx1, dv1, cev1)
                g1 = pltpu.async_copy(y_hbm.at[ridx1], rows1, sg1)
                if compute_w:
                    w1 = pltpu.async_copy(winv_hbm.at[cev1],
                                          wve.at[pl.ds(jj1 * CH, CH)], sw1)
                    w0.wait()
                    mask_w(jj0)
                g0.wait()
                scale(rows0, jj0)
                s0 = pltpu.async_copy(rows0, acc.at[dv0], ss, add=True)
                if compute_w:
                    w1.wait()
                    mask_w(jj1)
                g1.wait()
                scale(rows1, jj1)
                s1 = pltpu.async_copy(rows1, acc.at[dv1], ss, add=True)
                s0.wait()
                s1.wait()
                return carry2
            lax.fori_loop(0, BLK // 2, pair, 0)
            if compute_w:
                pltpu.sync_copy(wve, w_hbm.at[pl.ds(c * EP + ebase, BE)])
            return carry
        lax.fori_loop(0, CPT // BLK, block, 0)
        plsc.subcore_barrier()

        def outb(k, carry):
            bid = s + k * 16
            @pl.when(bid < NBB)
            def _():
                pltpu.sync_copy(acc.at[pl.ds(bid * NBS, NBS)],
                                o_hbm.at[pl.ds(c * NN + bid * NBS, NBS)])
            return carry
        lax.fori_loop(0, (NBB + 15) // 16, outb, 0)

    return _sc_edges


_sc_edges_l1 = _make_sc_edges(True)
_sc_edges_l2 = _make_sc_edges(False)


def _tc1_body(meta_ref, text_ref, mW1, mb1, mW2, mb2, tW1, tb1, tW2, tb2,
              g1W, g1root, g1b, cnt_ref,
              metaf_ref, textf_ref, y_ref, d_ref, inv_ref):
    mf = jnp.maximum(_dot(meta_ref[...], mW1[...].T) + mb1[...], 0.0)
    mf = jnp.maximum(_dot(mf, mW2[...].T) + mb2[...], 0.0)
    tf = jnp.maximum(_dot(text_ref[...], tW1[...].T) + tb1[...], 0.0)
    tf = jnp.maximum(_dot(tf, tW2[...].T) + tb2[...], 0.0)
    metaf_ref[...] = mf
    textf_ref[...] = tf
    for r in range(NR):
        yr = _dot(mf, g1W[r, :HH, :]) + _dot(tf, g1W[r, HH:, :])
        y_ref[0, r] = yr[:, :FP]
        y_ref[1, r] = yr[:, FP:]
    d = _dot(mf, g1root[:HH, :]) + _dot(tf, g1root[HH:, :]) + g1b[...]
    d_ref[0] = d[:, :FP]
    d_ref[1] = d[:, FP:]
    csum = cnt_ref[0, 0] + cnt_ref[1, 0]
    inv_ref[0] = 1.0 / jnp.maximum(csum, 1.0)


def _tc2_body(o1_ref, g2W, g2root, g2b, y_ref, d_ref):
    ra = jnp.maximum(o1_ref[0], 0.0)
    rb = jnp.maximum(o1_ref[1], 0.0)
    for r in range(NR):
        yr = _dot(ra, g2W[r, :FP, :]) + _dot(rb, g2W[r, FP:, :])
        y_ref[0, r] = yr[:, :FP]
        y_ref[1, r] = yr[:, FP:]
    d = _dot(ra, g2root[:FP, :]) + _dot(rb, g2root[FP:, :]) + g2b[...]
    d_ref[0] = d[:, :FP]
    d_ref[1] = d[:, FP:]


def _tc3_body(o2_ref, textf_ref, metaf_ref, Wcp, Wc, bi, Wo, bo, conWT, con_b,
              out_ref):
    ga = jnp.maximum(o2_ref[0], 0.0)
    gb = jnp.maximum(o2_ref[1], 0.0)
    tf = textf_ref[...]
    mf = metaf_ref[...]
    b3 = bi[...]

    Wcp_ = Wcp[...]
    Wc_ = Wc[...]
    pg = _dot(ga, Wcp_[:FP, :]) + _dot(gb, Wcp_[FP:, :]) + b3
    pt = _dot(tf, Wc_) + b3
    pm = _dot(mf, Wc_) + b3
    p3 = (pg, pt, pm)
    q = [p[:, 0:HH] for p in p3]
    kk = [p[:, HH:2 * HH] for p in p3]
    v = [p[:, 2 * HH:3 * HH] for p in p3]

    r_i = jax.lax.broadcasted_iota(jnp.int32, (3 * HH, 3 * NH), 0)
    c_i = jax.lax.broadcasted_iota(jnp.int32, (3 * HH, 3 * NH), 1)
    hselB = ((r_i // HH == c_i // NH) &
             ((r_i % HH) // DH == c_i % NH)).astype(jnp.float32)
    r_e = jax.lax.broadcasted_iota(jnp.int32, (3 * NH, 3 * HH), 0)
    c_e = jax.lax.broadcasted_iota(jnp.int32, (3 * NH, 3 * HH), 1)
    hselE = ((r_e // NH == c_e // HH) &
             (r_e % NH == (c_e % HH) // DH)).astype(jnp.float32)
    scale = 1.0 / (DH ** 0.5)

    vcat = jnp.concatenate([v[0], v[1], v[2]], axis=1)
    WoT = Wo[...].T
    fc = con_b[...]
    f_out = []
    for i in range(3):
        qk = jnp.concatenate([q[i] * kk[0], q[i] * kk[1], q[i] * kk[2]], axis=1)
        s12 = _dot(qk, hselB) * scale
        m = jnp.maximum(jnp.maximum(s12[:, 0:NH], s12[:, NH:2 * NH]),
                        s12[:, 2 * NH:3 * NH])
        mt = jnp.concatenate([m, m, m], axis=1)
        e = jnp.exp(s12 - mt)
        z = e[:, 0:NH] + e[:, NH:2 * NH] + e[:, 2 * NH:3 * NH]
        zt = jnp.concatenate([z, z, z], axis=1)
        a12 = e / zt
        ae = _dot(a12, hselE)
        prod = ae * vcat
        oi = prod[:, 0:HH] + prod[:, HH:2 * HH] + prod[:, 2 * HH:3 * HH]
        f_out.append(_dot(oi, WoT) + bo[...])
        for j in range(3):
            awij = jnp.sum(a12[:, j * NH:(j + 1) * NH], axis=1,
                           keepdims=True) * (1.0 / NH)
            fc = fc + awij * conWT[3 * i + j:3 * i + j + 1, :]

    out_ref[:, 0:HH] = f_out[0]
    out_ref[:, HH:2 * HH] = f_out[1]
    out_ref[:, 2 * HH:3 * HH] = f_out[2]
    out_ref[:, 3 * HH:4 * HH] = fc


def _full_spec(shape):
    return pl.BlockSpec(shape, lambda *args: tuple(0 for _ in shape))


def _pad48(w, axis):
    a = lax.slice_in_dim(w, 0, FH, axis=axis)
    b = lax.slice_in_dim(w, FH, HH, axis=axis)
    zshape = list(w.shape)
    zshape[axis] = FP - FH
    z = jnp.zeros(zshape, w.dtype)
    return jnp.concatenate([a, z, b, z], axis=axis)


def kernel(meta, text, edge_index, edge_type, meta_W1, meta_b1, meta_W2, meta_b2,
           text_W1, text_b1, text_W2, text_b2, g1_W, g1_root, g1_b,
           g2_W, g2_root, g2_b, attn_Wi, attn_bi, attn_Wo, attn_bo, con_W, con_b):
    f32 = jnp.float32
    ei = jnp.pad(edge_index, ((0, 0), (0, EP - EE)))
    src = ei[0]
    dst = ei[1]
    typ = jnp.pad(edge_type, (0, EP - EE))
    zeros_cb = jnp.zeros((CB,), f32)

    cnt = _sc_counts(dst, typ, zeros_cb)

    row2 = lambda x: x.reshape(1, -1)
    tc1 = pl.pallas_call(
        _tc1_body,
        grid=(NB,),
        in_specs=[
            pl.BlockSpec((BS, 16), lambda i: (i, 0)),
            pl.BlockSpec((BS, 768), lambda i: (i, 0)),
            _full_spec((HH, 16)), _full_spec((1, HH)),
            _full_spec((HH, HH)), _full_spec((1, HH)),
            _full_spec((HH, 768)), _full_spec((1, HH)),
            _full_spec((HH, HH)), _full_spec((1, HH)),
            _full_spec((NR, 2 * HH, 2 * FP)), _full_spec((2 * HH, 2 * FP)),
            _full_spec((1, 2 * FP)),
            pl.BlockSpec((2, 1, 1, CNTP // NB), lambda i: (0, i, 0, 0)),
        ],
        out_specs=[
            pl.BlockSpec((BS, HH), lambda i: (i, 0)),
            pl.BlockSpec((BS, HH), lambda i: (i, 0)),
            pl.BlockSpec((2, NR, BS, FP), lambda i: (0, 0, i, 0)),
            pl.BlockSpec((2, BS, FP), lambda i: (0, i, 0)),
            pl.BlockSpec((1, 1, CNTP // NB), lambda i: (i, 0, 0)),
        ],
        out_shape=[
            jax.ShapeDtypeStruct((NN, HH), f32),
            jax.ShapeDtypeStruct((NN, HH), f32),
            jax.ShapeDtypeStruct((2, NR, NN, FP), f32),
            jax.ShapeDtypeStruct((2, NN, FP), f32),
            jax.ShapeDtypeStruct((NB, 1, CNTP // NB), f32),
        ],
    )
    meta_f, text_f, y1, d1, inv = tc1(
        meta, text, meta_W1, row2(meta_b1), meta_W2, row2(meta_b2),
        text_W1, row2(text_b1), text_W2, row2(text_b2),
        _pad48(g1_W, 2), _pad48(g1_root, 1), _pad48(row2(g1_b), 1),
        cnt.reshape(2, NB, 1, CNTP // NB))

    inv_flat = inv.reshape(CNTP)
    o1, w_edges = _sc_edges_l1(src, dst, typ, y1.reshape(2 * NR * NN, FP),
                               inv_flat, d1.reshape(2 * NN, FP))

    tc2 = pl.pallas_call(
        _tc2_body,
        grid=(NB,),
        in_specs=[
            pl.BlockSpec((2, BS, FP), lambda i: (0, i, 0)),
            _full_spec((NR, 2 * FP, 2 * FP)), _full_spec((2 * FP, 2 * FP)),
            _full_spec((1, 2 * FP)),
        ],
        out_specs=[
            pl.BlockSpec((2, NR, BS, FP), lambda i: (0, 0, i, 0)),
            pl.BlockSpec((2, BS, FP), lambda i: (0, i, 0)),
        ],
        out_shape=[
            jax.ShapeDtypeStruct((2, NR, NN, FP), f32),
            jax.ShapeDtypeStruct((2, NN, FP), f32),
        ],
    )
    y2, d2 = tc2(o1.reshape(2, NN, FP),
                 _pad48(_pad48(g2_W, 1), 2),
                 _pad48(_pad48(g2_root, 0), 1),
                 _pad48(row2(g2_b), 1))

    o2 = _sc_edges_l2(src, dst, typ, y2.reshape(2 * NR * NN, FP), w_edges,
                      d2.reshape(2 * NN, FP))

    Wc = attn_Wi.T
    Wcp = _pad48(Wc, 0)
    tc3 = pl.pallas_call(
        _tc3_body,
        grid=(NB,),
        in_specs=[
            pl.BlockSpec((2, BS, FP), lambda i: (0, i, 0)),
            pl.BlockSpec((BS, HH), lambda i: (i, 0)),
            pl.BlockSpec((BS, HH), lambda i: (i, 0)),
            _full_spec((2 * FP, 3 * HH)), _full_spec((HH, 3 * HH)),
            _full_spec((1, 3 * HH)),
            _full_spec((HH, HH)), _full_spec((1, HH)),
            _full_spec((9, HH)), _full_spec((1, HH)),
        ],
        out_specs=pl.BlockSpec((BS, 4 * HH), lambda i: (i, 0)),
        out_shape=jax.ShapeDtypeStruct((NN, 4 * HH), f32),
    )
    out = tc3(o2.reshape(2, NN, FP), text_f, meta_f,
              Wcp, Wc, row2(attn_bi), attn_Wo, row2(attn_bo),
              con_W.T, row2(con_b))
    return out


# speedup vs baseline: 9.2838x; 1.0212x over previous
import functools

import jax
import jax.numpy as jnp
from jax import lax
from jax.experimental import pallas as pl
from jax.experimental.pallas import tpu as pltpu
from jax.experimental.pallas import tpu_sc as plsc

NN = 50000
EE = 800000
HH = 48
NR = 4
NH = 4
DH = HH // NH
FH = HH // 2
FP = 32
CH = 128
EP = 819200
NCHUNK = EP // CH
CPT = NCHUNK // 16
BLK = 20
BE = BLK * CH
BS = 2000
NB = NN // BS
NBS = 1000
NBB = NN // NBS
CNT = NR * NN
CNTP = 204800
CB = CNTP // 16

_prec = jax.lax.Precision.DEFAULT


def _dot(a, b):
    return jax.lax.dot_general(a, b, (((a.ndim - 1,), (0,)), ((), ())),
                               precision=_prec, preferred_element_type=jnp.float32)


_sc_mesh = plsc.VectorSubcoreMesh(core_axis_name="c", subcore_axis_name="s")


@functools.partial(
    pl.kernel,
    out_type=jax.ShapeDtypeStruct((2 * CNTP,), jnp.float32),
    mesh=_sc_mesh,
    scratch_types=[
        pltpu.VMEM_SHARED((CNTP,), jnp.float32),
        pltpu.VMEM((BE,), jnp.int32),
        pltpu.VMEM((BE,), jnp.int32),
        pltpu.VMEM((CH,), jnp.int32),
        pltpu.VMEM((CH,), jnp.float32),
    ],
)
def _sc_counts(dst_hbm, typ_hbm, zeros_hbm, cnt_hbm, cnt_sh, dve, tve, cev, val_v):
    c = lax.axis_index("c")
    s = lax.axis_index("s")
    w = c * 16 + s

    pltpu.sync_copy(zeros_hbm, cnt_sh.at[pl.ds(s * CB, CB)])
    plsc.subcore_barrier()

    iota = lax.iota(jnp.int32, 16)
    cpt = NCHUNK // 32

    def block(b, carry):
        ebase = w * (cpt * CH) + b * BE
        pltpu.sync_copy(dst_hbm.at[pl.ds(ebase, BE)], dve)
        pltpu.sync_copy(typ_hbm.at[pl.ds(ebase, BE)], tve)

        def chunk(j, carry2):
            off = j * CH
            for g in range(CH // 16):
                sl = pl.ds(off + g * 16, 16)
                gl = pl.ds(g * 16, 16)
                cev[gl] = tve[sl] * NN + dve[sl]
                ge = jnp.broadcast_to(ebase + off + g * 16, (16,)) + iota
                val_v[gl] = jnp.where(ge < EE, 1.0, 0.0)
            pltpu.sync_copy(val_v, cnt_sh.at[cev], add=True)
            return carry2
        lax.fori_loop(0, BLK, chunk, 0)
        return carry
    lax.fori_loop(0, cpt // BLK, block, 0)
    plsc.subcore_barrier()

    pltpu.sync_copy(cnt_sh.at[pl.ds(s * CB, CB)],
                    cnt_hbm.at[pl.ds(c * CNTP + s * CB, CB)])


def _make_sc_edges(compute_w):
    if compute_w:
        out_type = (jax.ShapeDtypeStruct((2 * NN, FP), jnp.float32),
                    jax.ShapeDtypeStruct((2 * EP,), jnp.float32))
    else:
        out_type = jax.ShapeDtypeStruct((2 * NN, FP), jnp.float32)

    ND = 4
    scratch = [
        pltpu.VMEM_SHARED((NN, FP), jnp.float32),
        pltpu.VMEM((BE,), jnp.int32),
        pltpu.VMEM((BE,), jnp.int32),
        pltpu.VMEM((BE,), jnp.int32),
        pltpu.VMEM((BE,), jnp.float32),
    ]
    scratch += [pltpu.VMEM((CH,), jnp.int32) for _ in range(ND)]
    scratch += [pltpu.VMEM((CH,), jnp.int32) for _ in range(ND)]
    scratch += [pltpu.VMEM((CH,), jnp.int32) for _ in range(ND)]
    scratch += [pltpu.VMEM((CH, FP), jnp.float32) for _ in range(ND)]
    scratch += [pltpu.SemaphoreType.DMA for _ in range(ND)]
    scratch += [pltpu.SemaphoreType.DMA for _ in range(ND)]
    scratch += [pltpu.SemaphoreType.DMA]

    @functools.partial(
        pl.kernel,
        out_type=out_type,
        mesh=_sc_mesh,
        scratch_types=scratch,
        compiler_params=pltpu.CompilerParams(use_tc_tiling_on_sc=False),
    )
    def _sc_edges(src_hbm, dst_hbm, typ_hbm, y_hbm, winv_hbm, dinit_hbm,
                  o_hbm, *rest):
        if compute_w:
            w_hbm = rest[0]
            rest = rest[1:]
        else:
            w_hbm = None
        acc, sve, dve, tve, wve = rest[0:5]
        ridx = rest[5:5 + ND]
        dv = rest[5 + ND:5 + 2 * ND]
        cev = rest[5 + 2 * ND:5 + 3 * ND]
        rows = rest[5 + 3 * ND:5 + 4 * ND]
        sg = rest[5 + 4 * ND:5 + 5 * ND]
        sw = rest[5 + 5 * ND:5 + 6 * ND]
        ssem = rest[5 + 6 * ND]
        c = lax.axis_index("c")
        s = lax.axis_index("s")
        coff = c * (NR * NN)
        iota = lax.iota(jnp.int32, 16)

        def initb(k, carry):
            bid = s + k * 16
            @pl.when(bid < NBB)
            def _():
                pltpu.sync_copy(dinit_hbm.at[pl.ds(c * NN + bid * NBS, NBS)],
                                acc.at[pl.ds(bid * NBS, NBS)])
            return carry
        lax.fori_loop(0, (NBB + 15) // 16, initb, 0)
        plsc.subcore_barrier()

        def block(b, carry):
            ebase = s * (CPT * CH) + b * BE
            pltpu.sync_copy(src_hbm.at[pl.ds(ebase, BE)], sve)
            pltpu.sync_copy(dst_hbm.at[pl.ds(ebase, BE)], dve)
            pltpu.sync_copy(typ_hbm.at[pl.ds(ebase, BE)], tve)
            if not compute_w:
                pltpu.sync_copy(winv_hbm.at[pl.ds(c * EP + ebase, BE)], wve)

            def do_idx(jj, t):
                off = jj * CH
                for g in range(CH // 16):
                    sl = pl.ds(off + g * 16, 16)
                    gl = pl.ds(g * 16, 16)
                    t16 = tve[sl]
                    ridx[t][gl] = coff + t16 * NN + sve[sl]
                    dv[t][gl] = dve[sl]
                    if compute_w:
                        cev[t][gl] = t16 * NN + dve[sl]

            def mask_w(jj):
                off = jj * CH
                for g in range(CH // 16):
                    sl = pl.ds(off + g * 16, 16)
                    ge = jnp.broadcast_to(ebase + off + g * 16, (16,)) + iota
                    wve[sl] = jnp.where(ge < EE, wve[sl], 0.0)

            def scale(t, jj):
                off = jj * CH
                for g in range(CH // 16):
                    wg = wve[pl.ds(off + g * 16, 16)]
                    for l in range(16):
                        i = g * 16 + l
                        wb = jnp.broadcast_to(wg[l], (16,))
                        for h in range(0, FP, 16):
                            rows[t][i, pl.ds(h, 16)] = (
                                rows[t][i, pl.ds(h, 16)] * wb)

            def quad(j4, carry2):
                jjs = [j4 * ND + t for t in range(ND)]
                gdesc = []
                wdesc = []
                for t in range(ND):
                    do_idx(jjs[t], t)
                    gdesc.append(
                        pltpu.async_copy(y_hbm.at[ridx[t]], rows[t], sg[t]))
                    if compute_w:
                        wdesc.append(pltpu.async_copy(
                            winv_hbm.at[cev[t]],
                            wve.at[pl.ds(jjs[t] * CH, CH)], sw[t]))
                sdesc = []
                for t in range(ND):
                    if compute_w:
                        wdesc[t].wait()
                        mask_w(jjs[t])
                    gdesc[t].wait()
                    scale(t, jjs[t])
                    sdesc.append(pltpu.async_copy(rows[t], acc.at[dv[t]],
                                                  ssem, add=True))
                for t in range(ND):
                    sdesc[t].wait()
                return carry2
            lax.fori_loop(0, BLK // ND, quad, 0)
            if compute_w:
                pltpu.sync_copy(wve, w_hbm.at[pl.ds(c * EP + ebase, BE)])
            return carry
        lax.fori_loop(0, CPT // BLK, block, 0)
        plsc.subcore_barrier()

        def outb(k, carry):
            bid = s + k * 16
            @pl.when(bid < NBB)
            def _():
                pltpu.sync_copy(acc.at[pl.ds(bid * NBS, NBS)],
                                o_hbm.at[pl.ds(c * NN + bid * NBS, NBS)])
            return carry
        lax.fori_loop(0, (NBB + 15) // 16, outb, 0)

    return _sc_edges


_sc_edges_l1 = _make_sc_edges(True)
_sc_edges_l2 = _make_sc_edges(False)


def _tc1_body(meta_ref, text_ref, mW1, mb1, mW2, mb2, tW1, tb1, tW2, tb2,
              g1W, g1root, g1b, cnt_ref,
              metaf_ref, textf_ref, y_ref, d_ref, inv_ref):
    mf = jnp.maximum(_dot(meta_ref[...], mW1[...].T) + mb1[...], 0.0)
    mf = jnp.maximum(_dot(mf, mW2[...].T) + mb2[...], 0.0)
    tf = jnp.maximum(_dot(text_ref[...], tW1[...].T) + tb1[...], 0.0)
    tf = jnp.maximum(_dot(tf, tW2[...].T) + tb2[...], 0.0)
    metaf_ref[...] = mf
    textf_ref[...] = tf
    for r in range(NR):
        yr = _dot(mf, g1W[r, :HH, :]) + _dot(tf, g1W[r, HH:, :])
        y_ref[0, r] = yr[:, :FP]
        y_ref[1, r] = yr[:, FP:]
    d = _dot(mf, g1root[:HH, :]) + _dot(tf, g1root[HH:, :]) + g1b[...]
    d_ref[0] = d[:, :FP]
    d_ref[1] = d[:, FP:]
    csum = cnt_ref[0, 0] + cnt_ref[1, 0]
    inv_ref[0] = 1.0 / jnp.maximum(csum, 1.0)


def _tc2_body(o1_ref, g2W, g2root, g2b, y_ref, d_ref):
    ra = jnp.maximum(o1_ref[0], 0.0)
    rb = jnp.maximum(o1_ref[1], 0.0)
    for r in range(NR):
        yr = _dot(ra, g2W[r, :FP, :]) + _dot(rb, g2W[r, FP:, :])
        y_ref[0, r] = yr[:, :FP]
        y_ref[1, r] = yr[:, FP:]
    d = _dot(ra, g2root[:FP, :]) + _dot(rb, g2root[FP:, :]) + g2b[...]
    d_ref[0] = d[:, :FP]
    d_ref[1] = d[:, FP:]


def _tc3_body(o2_ref, textf_ref, metaf_ref, Wcp, Wc, bi, Wo, bo, conWT, con_b,
              out_ref):
    ga = jnp.maximum(o2_ref[0], 0.0)
    gb = jnp.maximum(o2_ref[1], 0.0)
    tf = textf_ref[...]
    mf = metaf_ref[...]
    b3 = bi[...]

    Wcp_ = Wcp[...]
    Wc_ = Wc[...]
    pg = _dot(ga, Wcp_[:FP, :]) + _dot(gb, Wcp_[FP:, :]) + b3
    pt = _dot(tf, Wc_) + b3
    pm = _dot(mf, Wc_) + b3
    p3 = (pg, pt, pm)
    q = [p[:, 0:HH] for p in p3]
    kk = [p[:, HH:2 * HH] for p in p3]
    v = [p[:, 2 * HH:3 * HH] for p in p3]

    r_i = jax.lax.broadcasted_iota(jnp.int32, (3 * HH, 3 * NH), 0)
    c_i = jax.lax.broadcasted_iota(jnp.int32, (3 * HH, 3 * NH), 1)
    hselB = ((r_i // HH == c_i // NH) &
             ((r_i % HH) // DH == c_i % NH)).astype(jnp.float32)
    r_e = jax.lax.broadcasted_iota(jnp.int32, (3 * NH, 3 * HH), 0)
    c_e = jax.lax.broadcasted_iota(jnp.int32, (3 * NH, 3 * HH), 1)
    hselE = ((r_e // NH == c_e // HH) &
             (r_e % NH == (c_e % HH) // DH)).astype(jnp.float32)
    scale = 1.0 / (DH ** 0.5)

    vcat = jnp.concatenate([v[0], v[1], v[2]], axis=1)
    WoT = Wo[...].T
    fc = con_b[...]
    f_out = []
    for i in range(3):
        qk = jnp.concatenate([q[i] * kk[0], q[i] * kk[1], q[i] * kk[2]], axis=1)
        s12 = _dot(qk, hselB) * scale
        m = jnp.maximum(jnp.maximum(s12[:, 0:NH], s12[:, NH:2 * NH]),
                        s12[:, 2 * NH:3 * NH])
        mt = jnp.concatenate([m, m, m], axis=1)
        e = jnp.exp(s12 - mt)
        z = e[:, 0:NH] + e[:, NH:2 * NH] + e[:, 2 * NH:3 * NH]
        zt = jnp.concatenate([z, z, z], axis=1)
        a12 = e / zt
        ae = _dot(a12, hselE)
        prod = ae * vcat
        oi = prod[:, 0:HH] + prod[:, HH:2 * HH] + prod[:, 2 * HH:3 * HH]
        f_out.append(_dot(oi, WoT) + bo[...])
        for j in range(3):
            awij = jnp.sum(a12[:, j * NH:(j + 1) * NH], axis=1,
                           keepdims=True) * (1.0 / NH)
            fc = fc + awij * conWT[3 * i + j:3 * i + j + 1, :]

    out_ref[:, 0:HH] = f_out[0]
    out_ref[:, HH:2 * HH] = f_out[1]
    out_ref[:, 2 * HH:3 * HH] = f_out[2]
    out_ref[:, 3 * HH:4 * HH] = fc


def _full_spec(shape):
    return pl.BlockSpec(shape, lambda *args: tuple(0 for _ in shape))


def _pad48(w, axis):
    a = lax.slice_in_dim(w, 0, FH, axis=axis)
    b = lax.slice_in_dim(w, FH, HH, axis=axis)
    zshape = list(w.shape)
    zshape[axis] = FP - FH
    z = jnp.zeros(zshape, w.dtype)
    return jnp.concatenate([a, z, b, z], axis=axis)


def kernel(meta, text, edge_index, edge_type, meta_W1, meta_b1, meta_W2, meta_b2,
           text_W1, text_b1, text_W2, text_b2, g1_W, g1_root, g1_b,
           g2_W, g2_root, g2_b, attn_Wi, attn_bi, attn_Wo, attn_bo, con_W, con_b):
    f32 = jnp.float32
    ei = jnp.pad(edge_index, ((0, 0), (0, EP - EE)))
    src = ei[0]
    dst = ei[1]
    typ = jnp.pad(edge_type, (0, EP - EE))
    zeros_cb = jnp.zeros((CB,), f32)

    cnt = _sc_counts(dst, typ, zeros_cb)

    row2 = lambda x: x.reshape(1, -1)
    tc1 = pl.pallas_call(
        _tc1_body,
        grid=(NB,),
        in_specs=[
            pl.BlockSpec((BS, 16), lambda i: (i, 0)),
            pl.BlockSpec((BS, 768), lambda i: (i, 0)),
            _full_spec((HH, 16)), _full_spec((1, HH)),
            _full_spec((HH, HH)), _full_spec((1, HH)),
            _full_spec((HH, 768)), _full_spec((1, HH)),
            _full_spec((HH, HH)), _full_spec((1, HH)),
            _full_spec((NR, 2 * HH, 2 * FP)), _full_spec((2 * HH, 2 * FP)),
            _full_spec((1, 2 * FP)),
            pl.BlockSpec((2, 1, 1, CNTP // NB), lambda i: (0, i, 0, 0)),
        ],
        out_specs=[
            pl.BlockSpec((BS, HH), lambda i: (i, 0)),
            pl.BlockSpec((BS, HH), lambda i: (i, 0)),
            pl.BlockSpec((2, NR, BS, FP), lambda i: (0, 0, i, 0)),
            pl.BlockSpec((2, BS, FP), lambda i: (0, i, 0)),
            pl.BlockSpec((1, 1, CNTP // NB), lambda i: (i, 0, 0)),
        ],
        out_shape=[
            jax.ShapeDtypeStruct((NN, HH), f32),
            jax.ShapeDtypeStruct((NN, HH), f32),
            jax.ShapeDtypeStruct((2, NR, NN, FP), f32),
            jax.ShapeDtypeStruct((2, NN, FP), f32),
            jax.ShapeDtypeStruct((NB, 1, CNTP // NB), f32),
        ],
    )
    meta_f, text_f, y1, d1, inv = tc1(
        meta, text, meta_W1, row2(meta_b1), meta_W2, row2(meta_b2),
        text_W1, row2(text_b1), text_W2, row2(text_b2),
        _pad48(g1_W, 2), _pad48(g1_root, 1), _pad48(row2(g1_b), 1),
        cnt.reshape(2, NB, 1, CNTP // NB))

    inv_flat = inv.reshape(CNTP)
    o1, w_edges = _sc_edges_l1(src, dst, typ, y1.reshape(2 * NR * NN, FP),
                               inv_flat, d1.reshape(2 * NN, FP))

    tc2 = pl.pallas_call(
        _tc2_body,
        grid=(NB,),
        in_specs=[
            pl.BlockSpec((2, BS, FP), lambda i: (0, i, 0)),
            _full_spec((NR, 2 * FP, 2 * FP)), _full_spec((2 * FP, 2 * FP)),
            _full_spec((1, 2 * FP)),
        ],
        out_specs=[
            pl.BlockSpec((2, NR, BS, FP), lambda i: (0, 0, i, 0)),
            pl.BlockSpec((2, BS, FP), lambda i: (0, i, 0)),
        ],
        out_shape=[
            jax.ShapeDtypeStruct((2, NR, NN, FP), f32),
            jax.ShapeDtypeStruct((2, NN, FP), f32),
        ],
    )
    y2, d2 = tc2(o1.reshape(2, NN, FP),
                 _pad48(_pad48(g2_W, 1), 2),
                 _pad48(_pad48(g2_root, 0), 1),
                 _pad48(row2(g2_b), 1))

    o2 = _sc_edges_l2(src, dst, typ, y2.reshape(2 * NR * NN, FP), w_edges,
                      d2.reshape(2 * NN, FP))

    Wc = attn_Wi.T
    Wcp = _pad48(Wc, 0)
    tc3 = pl.pallas_call(
        _tc3_body,
        grid=(NB,),
        in_specs=[
            pl.BlockSpec((2, BS, FP), lambda i: (0, i, 0)),
            pl.BlockSpec((BS, HH), lambda i: (i, 0)),
            pl.BlockSpec((BS, HH), lambda i: (i, 0)),
            _full_spec((2 * FP, 3 * HH)), _full_spec((HH, 3 * HH)),
            _full_spec((1, 3 * HH)),
            _full_spec((HH, HH)), _full_spec((1, HH)),
            _full_spec((9, HH)), _full_spec((1, HH)),
        ],
        out_specs=pl.BlockSpec((BS, 4 * HH), lambda i: (i, 0)),
        out_shape=jax.ShapeDtypeStruct((NN, 4 * HH), f32),
    )
    out = tc3(o2.reshape(2, NN, FP), text_f, meta_f,
              Wcp, Wc, row2(attn_bi), attn_Wo, row2(attn_bo),
              con_W.T, row2(con_b))
    return out
